# Initial kernel scaffold; baseline (speedup 1.0000x reference)
import jax
import jax.numpy as jnp
from jax.experimental import pallas as pl


def kernel(h, edge_index, W, a):
    raise NotImplementedError("write your pallas kernel here")


# speedup vs baseline: 11.9426x; 11.9426x over previous
import functools

import jax
import jax.numpy as jnp
from jax import lax
from jax.experimental import pallas as pl
from jax.experimental.pallas import tpu as pltpu
from jax.experimental.pallas import tpu_sc as plsc

N = 10000
D = 128
E = 320000
NCORES = 2
NSUB = 16
NW = NCORES * NSUB
EPT = E // NW
C = 16
SEG = 2000
NSEG = EPT // SEG
NCHUNK = SEG // C
N_PAD = 10240
RPT = N_PAD // NSUB
DEN_PAD = 10240
NBLK = 10
BLK = N // NBLK


def _tc_front(h_ref, w_ref, a_ref, z_ref, s1_ref, s2_ref, m1_ref, m2_ref):
    hb = h_ref[...]
    zb = lax.dot_general(hb, w_ref[...], (((1,), (1,)), ((), ())),
                         preferred_element_type=jnp.float32)
    z_ref[...] = zb
    a1 = a_ref[0, :D]
    a2 = a_ref[0, D:]
    s1 = jnp.sum(zb * a1[None, :], axis=1)
    s2 = jnp.sum(zb * a2[None, :], axis=1)
    s1_ref[0, 0, :] = s1
    s2_ref[0, 0, :] = s2
    m1_ref[0, 0, :] = jnp.full((D,), jnp.max(s1), jnp.float32)
    m2_ref[0, 0, :] = jnp.full((D,), jnp.max(s2), jnp.float32)


_tc_front_call = functools.partial(
    pl.pallas_call,
    grid=(NBLK,),
    in_specs=[
        pl.BlockSpec((BLK, D), lambda i: (i, 0)),
        pl.BlockSpec((D, D), lambda i: (0, 0)),
        pl.BlockSpec((1, 2 * D), lambda i: (0, 0)),
    ],
    out_specs=[
        pl.BlockSpec((BLK, D), lambda i: (i, 0)),
        pl.BlockSpec((1, 1, BLK), lambda i: (i, 0, 0)),
        pl.BlockSpec((1, 1, BLK), lambda i: (i, 0, 0)),
        pl.BlockSpec((1, 1, D), lambda i: (i, 0, 0)),
        pl.BlockSpec((1, 1, D), lambda i: (i, 0, 0)),
    ],
    out_shape=[
        jax.ShapeDtypeStruct((N, D), jnp.float32),
        jax.ShapeDtypeStruct((NBLK, 1, BLK), jnp.float32),
        jax.ShapeDtypeStruct((NBLK, 1, BLK), jnp.float32),
        jax.ShapeDtypeStruct((NBLK, 1, D), jnp.float32),
        jax.ShapeDtypeStruct((NBLK, 1, D), jnp.float32),
    ],
)(_tc_front)


_sc_mesh = plsc.VectorSubcoreMesh(core_axis_name="c", subcore_axis_name="s")


@functools.partial(
    pl.kernel,
    out_type=[
        jax.ShapeDtypeStruct((NCORES, N_PAD, D), jnp.float32),
        jax.ShapeDtypeStruct((NW, DEN_PAD), jnp.float32),
    ],
    mesh=_sc_mesh,
    compiler_params=pltpu.CompilerParams(needs_layout_passes=False),
    scratch_types=[
        pltpu.VMEM((SEG,), jnp.int32),
        pltpu.VMEM((SEG,), jnp.int32),
        pltpu.VMEM((N,), jnp.float32),
        pltpu.VMEM((N,), jnp.float32),
        pltpu.VMEM((DEN_PAD,), jnp.float32),
        pltpu.VMEM((C, D), jnp.float32),
        pltpu.VMEM((16,), jnp.float32),
        pltpu.VMEM_SHARED((N_PAD, D), jnp.float32),
        pltpu.SemaphoreType.DMA,
    ],
)
def _sc_edges(z_hbm, src_hbm, dst_hbm, s1_hbm, s2_hbm, m_hbm, zz_hbm, z1_hbm,
              nump_hbm, denp_hbm,
              src_v, dst_v, s1_v, s2_v, den_v, rows_v, m_v, num_sh, sem):
    cid = lax.axis_index("c")
    sid = lax.axis_index("s")
    wid = sid * NCORES + cid
    base = pl.multiple_of(wid * EPT, 8)

    pltpu.sync_copy(s1_hbm, s1_v)
    pltpu.sync_copy(s2_hbm, s2_v)
    pltpu.sync_copy(m_hbm, m_v)
    pltpu.sync_copy(z1_hbm, den_v)
    row0 = pl.multiple_of(sid * RPT, 8)
    pltpu.sync_copy(zz_hbm.at[pl.ds(row0, RPT)], num_sh.at[pl.ds(row0, RPT)])
    plsc.subcore_barrier()

    def seg_body(g, carry):
        segbase = pl.multiple_of(base + g * SEG, 8)
        pltpu.sync_copy(src_hbm.at[pl.ds(segbase, SEG)], src_v)
        pltpu.sync_copy(dst_hbm.at[pl.ds(segbase, SEG)], dst_v)
        lax.fori_loop(0, NCHUNK, body, 0)
        return carry

    def body(i, carry):
        off = pl.multiple_of(i * C, C)
        src16 = src_v[pl.ds(off, C)]
        dst16 = dst_v[pl.ds(off, C)]
        g = plsc.load_gather(s1_v, [src16]) + plsc.load_gather(s2_v, [dst16])
        e = jnp.where(g >= 0, g, g * jnp.float32(0.01))
        ex = jnp.exp(e - m_v[...])
        pltpu.async_copy(z_hbm.at[src16], rows_v, sem).wait()
        dnums = lax.GatherDimensionNumbers(
            offset_dims=(), collapsed_slice_dims=(0,), start_index_map=(0,))
        for r in range(C):
            sp = lax.gather(ex, jnp.full((16, 1), r, jnp.int32), dnums, (1,),
                            mode=lax.GatherScatterMode.PROMISE_IN_BOUNDS)
            row = rows_v.at[r]
            for j in range(D // 16):
                cs = pl.ds(j * 16, 16)
                row[cs] = row[cs] * sp
        pltpu.sync_copy(rows_v, num_sh.at[dst16], add=True)
        plsc.addupdate_scatter(den_v, [dst16], ex)
        return carry

    lax.fori_loop(0, NSEG, seg_body, 0)
    plsc.subcore_barrier()

    pltpu.sync_copy(den_v, denp_hbm.at[wid])
    pltpu.sync_copy(num_sh.at[pl.ds(row0, RPT)],
                    nump_hbm.at[cid, pl.ds(row0, RPT)])


def _tc_combine(nump_ref, denp_ref, out_ref):
    num = nump_ref[0] + nump_ref[1]
    den = jnp.sum(denp_ref[...], axis=0)
    safe = den > 0
    inv = safe.astype(jnp.float32) / jnp.where(safe, den, jnp.float32(1.0))
    out_ref[...] = num * inv[:, None]


_tc_combine_call = functools.partial(
    pl.pallas_call,
    grid=(NBLK,),
    in_specs=[
        pl.BlockSpec((NCORES, 1024, D), lambda i: (0, i, 0)),
        pl.BlockSpec((NW, 1024), lambda i: (0, i)),
    ],
    out_specs=pl.BlockSpec((1024, D), lambda i: (i, 0)),
    out_shape=jax.ShapeDtypeStruct((N, D), jnp.float32),
)(_tc_combine)


def kernel(h, edge_index, W, a):
    z, s1, s2, m1, m2 = _tc_front_call(h, W, a)
    s1 = s1.reshape(N)
    s2 = s2.reshape(N)
    msum = jnp.max(m1) + jnp.max(m2)
    mglob = jnp.where(msum >= 0, msum, msum * jnp.float32(0.01))
    m16 = jnp.full((16,), mglob, jnp.float32)
    src = edge_index[0]
    dst = edge_index[1]
    zz = jnp.zeros((N_PAD, D), jnp.float32)
    z1 = jnp.zeros((DEN_PAD,), jnp.float32)
    nump, denp = _sc_edges(z, src, dst, s1, s2, m16, zz, z1)
    return _tc_combine_call(nump, denp)


# speedup vs baseline: 22.2199x; 1.8606x over previous
import functools

import jax
import jax.numpy as jnp
from jax import lax
from jax.experimental import pallas as pl
from jax.experimental.pallas import tpu as pltpu
from jax.experimental.pallas import tpu_sc as plsc

N = 10000
D = 128
E = 320000
NCORES = 2
NSUB = 16
NW = NCORES * NSUB
EPT = E // NW
C = 80
SEG = 2000
NSEG = EPT // SEG
NCHUNK = SEG // C
CROWS = EPT // C
N_PAD = 10240
RPT = N_PAD // NSUB
DEN_PAD = 10240
NBLK = 10
BLK = N // NBLK


def _tc_front(h_ref, w_ref, a_ref, z_ref, s1_ref, s2_ref, m1_ref, m2_ref):
    hb = h_ref[...]
    zb = lax.dot_general(hb, w_ref[...], (((1,), (1,)), ((), ())),
                         preferred_element_type=jnp.float32)
    z_ref[...] = zb
    a1 = a_ref[0, :D]
    a2 = a_ref[0, D:]
    s1 = jnp.sum(zb * a1[None, :], axis=1)
    s2 = jnp.sum(zb * a2[None, :], axis=1)
    s1_ref[0, 0, :] = s1
    s2_ref[0, 0, :] = s2
    m1_ref[0, 0, :] = jnp.full((D,), jnp.max(s1), jnp.float32)
    m2_ref[0, 0, :] = jnp.full((D,), jnp.max(s2), jnp.float32)


_tc_front_call = functools.partial(
    pl.pallas_call,
    grid=(NBLK,),
    in_specs=[
        pl.BlockSpec((BLK, D), lambda i: (i, 0)),
        pl.BlockSpec((D, D), lambda i: (0, 0)),
        pl.BlockSpec((1, 2 * D), lambda i: (0, 0)),
    ],
    out_specs=[
        pl.BlockSpec((BLK, D), lambda i: (i, 0)),
        pl.BlockSpec((1, 1, BLK), lambda i: (i, 0, 0)),
        pl.BlockSpec((1, 1, BLK), lambda i: (i, 0, 0)),
        pl.BlockSpec((1, 1, D), lambda i: (i, 0, 0)),
        pl.BlockSpec((1, 1, D), lambda i: (i, 0, 0)),
    ],
    out_shape=[
        jax.ShapeDtypeStruct((N, D), jnp.float32),
        jax.ShapeDtypeStruct((NBLK, 1, BLK), jnp.float32),
        jax.ShapeDtypeStruct((NBLK, 1, BLK), jnp.float32),
        jax.ShapeDtypeStruct((NBLK, 1, D), jnp.float32),
        jax.ShapeDtypeStruct((NBLK, 1, D), jnp.float32),
    ],
)(_tc_front)


_sc_mesh = plsc.VectorSubcoreMesh(core_axis_name="c", subcore_axis_name="s")


@functools.partial(
    pl.kernel,
    out_type=[
        jax.ShapeDtypeStruct((NCORES, N_PAD, D), jnp.float32),
        jax.ShapeDtypeStruct((NW, DEN_PAD), jnp.float32),
    ],
    mesh=_sc_mesh,
    compiler_params=pltpu.CompilerParams(needs_layout_passes=False),
    scratch_types=[
        pltpu.VMEM((NCHUNK, C), jnp.int32),
        pltpu.VMEM((NCHUNK, C), jnp.int32),
        pltpu.VMEM((N,), jnp.float32),
        pltpu.VMEM((N,), jnp.float32),
        pltpu.VMEM((DEN_PAD,), jnp.float32),
        pltpu.VMEM((C, D), jnp.float32),
        pltpu.VMEM((16,), jnp.float32),
        pltpu.VMEM_SHARED((N_PAD, D), jnp.float32),
        pltpu.SemaphoreType.DMA,
    ],
)
def _sc_edges(z_hbm, src_hbm, dst_hbm, s1_hbm, s2_hbm, m_hbm, zz_hbm, z1_hbm,
              nump_hbm, denp_hbm,
              src_v, dst_v, s1_v, s2_v, den_v, rows_v, m_v, num_sh, sem):
    cid = lax.axis_index("c")
    sid = lax.axis_index("s")
    wid = sid * NCORES + cid
    base = pl.multiple_of(wid * EPT, 8)

    pltpu.sync_copy(s1_hbm, s1_v)
    pltpu.sync_copy(s2_hbm, s2_v)
    pltpu.sync_copy(m_hbm, m_v)
    pltpu.sync_copy(z1_hbm, den_v)
    row0 = pl.multiple_of(sid * RPT, 8)
    pltpu.sync_copy(zz_hbm.at[pl.ds(row0, RPT)], num_sh.at[pl.ds(row0, RPT)])
    plsc.subcore_barrier()

    dnums = lax.GatherDimensionNumbers(
        offset_dims=(), collapsed_slice_dims=(0,), start_index_map=(0,))

    def seg_body(g, carry):
        pltpu.sync_copy(src_hbm.at[wid, g], src_v)
        pltpu.sync_copy(dst_hbm.at[wid, g], dst_v)
        lax.fori_loop(0, NCHUNK, body, 0)
        return carry

    def body(i, carry):
        srow = src_v.at[i]
        drow = dst_v.at[i]
        pltpu.async_copy(z_hbm.at[srow], rows_v, sem).wait()
        mvec = m_v[...]
        for k in range(C // 16):
            sl = pl.ds(k * 16, 16)
            s16 = srow[sl]
            d16 = drow[sl]
            g = plsc.load_gather(s1_v, [s16]) + plsc.load_gather(s2_v, [d16])
            e = jnp.where(g >= 0, g, g * jnp.float32(0.01))
            ex = jnp.exp(e - mvec)
            for r in range(16):
                sp = lax.gather(ex, jnp.full((16, 1), r, jnp.int32), dnums,
                                (1,),
                                mode=lax.GatherScatterMode.PROMISE_IN_BOUNDS)
                row = rows_v.at[k * 16 + r]
                for j in range(D // 16):
                    cs = pl.ds(j * 16, 16)
                    row[cs] = row[cs] * sp
            plsc.addupdate_scatter(den_v, [d16], ex)
        pltpu.sync_copy(rows_v, num_sh.at[drow], add=True)
        return carry

    lax.fori_loop(0, NSEG, seg_body, 0)
    plsc.subcore_barrier()

    pltpu.sync_copy(den_v, denp_hbm.at[wid])
    pltpu.sync_copy(num_sh.at[pl.ds(row0, RPT)],
                    nump_hbm.at[cid, pl.ds(row0, RPT)])


def _tc_combine(nump_ref, denp_ref, out_ref):
    num = nump_ref[0] + nump_ref[1]
    den = jnp.sum(denp_ref[...], axis=0)
    safe = den > 0
    inv = safe.astype(jnp.float32) / jnp.where(safe, den, jnp.float32(1.0))
    out_ref[...] = num * inv[:, None]


_tc_combine_call = functools.partial(
    pl.pallas_call,
    grid=(NBLK,),
    in_specs=[
        pl.BlockSpec((NCORES, 1024, D), lambda i: (0, i, 0)),
        pl.BlockSpec((NW, 1024), lambda i: (0, i)),
    ],
    out_specs=pl.BlockSpec((1024, D), lambda i: (i, 0)),
    out_shape=jax.ShapeDtypeStruct((N, D), jnp.float32),
)(_tc_combine)


def kernel(h, edge_index, W, a):
    z, s1, s2, m1, m2 = _tc_front_call(h, W, a)
    s1 = s1.reshape(N)
    s2 = s2.reshape(N)
    msum = jnp.max(m1) + jnp.max(m2)
    mglob = jnp.where(msum >= 0, msum, msum * jnp.float32(0.01))
    m16 = jnp.full((16,), mglob, jnp.float32)
    src = edge_index[0].reshape(NW, NSEG, NCHUNK, C)
    dst = edge_index[1].reshape(NW, NSEG, NCHUNK, C)
    zz = jnp.zeros((N_PAD, D), jnp.float32)
    z1 = jnp.zeros((DEN_PAD,), jnp.float32)
    nump, denp = _sc_edges(z, src, dst, s1, s2, m16, zz, z1)
    return _tc_combine_call(nump, denp)
